# Initial kernel scaffold; baseline (speedup 1.0000x reference)
import jax
import jax.numpy as jnp
from jax.experimental import pallas as pl


def kernel(x, edge_index, batch, Wl1, bl1, Wr1, Wl2, bl2, Wr2, Wfc1, bfc1, Wfc2, bfc2):
    raise NotImplementedError("write your pallas kernel here")


# speedup vs baseline: 4.7622x; 4.7622x over previous
import jax
import jax.numpy as jnp
from jax import lax
from jax.experimental import pallas as pl
from jax.experimental.pallas import tpu as pltpu
from jax.experimental.pallas import tpu_sc as plsc

N = 10000
E = 320000
D = 128
C = 10
G = 64

NC = 2
NS = 16
NW = NC * NS
EP = E // NW
CH = 80
NCH = EP // CH

WSTR = 624
TAIL = 16
TOFF = NS * WSTR
ZCH = 48
NZC = WSTR // ZCH

BLK = 2000
PBLK = 1000
NPB = N // PBLK


def _stage_zero(z_h, zb_v, acc_s, sid):
    pltpu.sync_copy(z_h, zb_v)
    for k in range(NZC):
        pltpu.sync_copy(zb_v, acc_s.at[pl.ds(sid * WSTR + k * ZCH, ZCH)])

    @pl.when(sid == 0)
    def _():
        pltpu.sync_copy(zb_v.at[pl.ds(0, TAIL)], acc_s.at[pl.ds(TOFF, TAIL)])


def _stage_out(acc_s, zb_v, out_h, cid, sid):
    for k in range(NZC):
        r0 = sid * WSTR + k * ZCH
        pltpu.sync_copy(acc_s.at[pl.ds(r0, ZCH)], zb_v)
        pltpu.sync_copy(zb_v, out_h.at[pl.ds(cid * N + r0, ZCH)])

    @pl.when(sid == 0)
    def _():
        pltpu.sync_copy(acc_s.at[pl.ds(TOFF, TAIL)], zb_v.at[pl.ds(0, TAIL)])
        pltpu.sync_copy(zb_v.at[pl.ds(0, TAIL)],
                        out_h.at[pl.ds(cid * N + TOFF, TAIL)])


def _sc_body_deg(dst_h, ones_h, zeros_h, degw_h,
                 acc_s, dst_v, ones_v, zb_v):
    cid = lax.axis_index("c")
    sid = lax.axis_index("s")
    wid = cid * NS + sid

    _stage_zero(zeros_h, zb_v, acc_s, sid)
    pltpu.sync_copy(ones_h, ones_v)
    plsc.subcore_barrier()

    base = wid * EP

    def _chunk(k, carry):
        off = base + k * CH
        pltpu.sync_copy(dst_h.at[pl.ds(off, CH)], dst_v)
        pltpu.sync_copy(ones_v, acc_s.at[dst_v], add=True)
        return carry
    lax.fori_loop(0, NCH, _chunk, 0)

    plsc.subcore_barrier()
    _stage_out(acc_s, zb_v, degw_h, cid, sid)


def _sc_body_feat(x_h, src_h, dst_h, zeros_h, parts_h,
                  acc_s, src_v, dst_v, rows_v, zb_v, sem):
    cid = lax.axis_index("c")
    sid = lax.axis_index("s")
    wid = cid * NS + sid

    _stage_zero(zeros_h, zb_v, acc_s, sid)
    plsc.subcore_barrier()

    base = wid * EP

    def _chunk(k, carry):
        off = base + k * CH
        pltpu.sync_copy(src_h.at[pl.ds(off, CH)], src_v)
        pltpu.sync_copy(dst_h.at[pl.ds(off, CH)], dst_v)
        pltpu.async_copy(x_h.at[src_v], rows_v, sem).wait()
        pltpu.sync_copy(rows_v, acc_s.at[dst_v], add=True)
        return carry
    lax.fori_loop(0, NCH, _chunk, 0)

    plsc.subcore_barrier()
    _stage_out(acc_s, zb_v, parts_h, cid, sid)


def _sc_deg(dst, ones, zeros):
    mesh = plsc.VectorSubcoreMesh(core_axis_name="c", subcore_axis_name="s")
    f = pl.kernel(
        _sc_body_deg,
        out_type=jax.ShapeDtypeStruct((NC * N, D), jnp.float32),
        mesh=mesh,
        scratch_types=[
            pltpu.VMEM_SHARED((N, D), jnp.float32),
            pltpu.VMEM((CH,), jnp.int32),
            pltpu.VMEM((CH, D), jnp.float32),
            pltpu.VMEM((ZCH, D), jnp.float32),
        ],
    )
    return f(dst, ones, zeros)


def _sc_feat(x, src, dst, zeros):
    mesh = plsc.VectorSubcoreMesh(core_axis_name="c", subcore_axis_name="s")
    f = pl.kernel(
        _sc_body_feat,
        out_type=jax.ShapeDtypeStruct((NC * N, D), jnp.float32),
        mesh=mesh,
        scratch_types=[
            pltpu.VMEM_SHARED((N, D), jnp.float32),
            pltpu.VMEM((CH,), jnp.int32),
            pltpu.VMEM((CH,), jnp.int32),
            pltpu.VMEM((CH, D), jnp.float32),
            pltpu.VMEM((ZCH, D), jnp.float32),
            pltpu.SemaphoreType.DMA,
        ],
    )
    return f(x, src, dst, zeros)


def _combine_body(parts_ref, degw_ref, x_ref, wl_ref, bl_ref, wr_ref, out_ref):
    s = parts_ref[0] + parts_ref[1]
    deg = degw_ref[0] + degw_ref[1]
    agg = s / jnp.maximum(deg, 1.0)
    out_ref[...] = (
        lax.dot_general(agg, wl_ref[...], (((1,), (1,)), ((), ())),
                        preferred_element_type=jnp.float32)
        + lax.dot_general(x_ref[...], wr_ref[...], (((1,), (1,)), ((), ())),
                          preferred_element_type=jnp.float32)
        + bl_ref[...]
    )


def _tc_combine(parts, degw, x, wl, bl, wr):
    grid = (N // BLK,)
    return pl.pallas_call(
        _combine_body,
        grid=grid,
        in_specs=[
            pl.BlockSpec((NC, BLK, D), lambda i: (0, i, 0)),
            pl.BlockSpec((NC, BLK, D), lambda i: (0, i, 0)),
            pl.BlockSpec((BLK, D), lambda i: (i, 0)),
            pl.BlockSpec((D, D), lambda i: (0, 0)),
            pl.BlockSpec((1, D), lambda i: (0, 0)),
            pl.BlockSpec((D, D), lambda i: (0, 0)),
        ],
        out_specs=pl.BlockSpec((BLK, D), lambda i: (i, 0)),
        out_shape=jax.ShapeDtypeStruct((N, D), jnp.float32),
    )(parts, degw, x, wl, bl, wr)


def _pool_body(x1_ref, x2_ref, b_ref, wf1_ref, bf1_ref, wf2_ref, bf2_ref,
               out_ref, acc_ref):
    i = pl.program_id(0)

    @pl.when(i == 0)
    def _():
        acc_ref[...] = jnp.full((G, 2 * D), -jnp.inf, jnp.float32)

    bv = b_ref[0]
    b_lo = jnp.min(bv)
    b_hi = jnp.max(bv)
    h1 = x1_ref[...]
    h2 = x2_ref[...]
    neg = jnp.float32(-jnp.inf)

    def _seg(b, carry):
        m = bv == b
        m1 = jnp.max(jnp.where(m, h1, neg), axis=0)
        m2 = jnp.max(jnp.where(m, h2, neg), axis=0)
        row = jnp.concatenate([m1, m2])[None, :]
        acc_ref[pl.ds(b, 1), :] = jnp.maximum(acc_ref[pl.ds(b, 1), :], row)
        return carry
    lax.fori_loop(b_lo, b_hi + 1, _seg, 0)

    @pl.when(i == NPB - 1)
    def _():
        pooled = acc_ref[...]
        z = lax.dot_general(pooled, wf1_ref[...], (((1,), (1,)), ((), ())),
                            preferred_element_type=jnp.float32) + bf1_ref[...]
        z = jnp.maximum(z, 0.0)
        logits = lax.dot_general(z, wf2_ref[...], (((1,), (1,)), ((), ())),
                                 preferred_element_type=jnp.float32) + bf2_ref[...]
        mx = jnp.max(logits, axis=-1, keepdims=True)
        sh = logits - mx
        lse = jnp.log(jnp.sum(jnp.exp(sh), axis=-1, keepdims=True))
        out_ref[...] = sh - lse


def _pool_mlp(x1, x2, batch3d, wf1, bf1, wf2, bf2):
    grid = (NPB,)
    return pl.pallas_call(
        _pool_body,
        grid=grid,
        in_specs=[
            pl.BlockSpec((PBLK, D), lambda i: (i, 0)),
            pl.BlockSpec((PBLK, D), lambda i: (i, 0)),
            pl.BlockSpec((1, PBLK, 1), lambda i: (i, 0, 0)),
            pl.BlockSpec((D, 2 * D), lambda i: (0, 0)),
            pl.BlockSpec((1, D), lambda i: (0, 0)),
            pl.BlockSpec((C, D), lambda i: (0, 0)),
            pl.BlockSpec((1, C), lambda i: (0, 0)),
        ],
        out_specs=pl.BlockSpec((G, C), lambda i: (0, 0)),
        out_shape=jax.ShapeDtypeStruct((G, C), jnp.float32),
        scratch_shapes=[pltpu.VMEM((G, 2 * D), jnp.float32)],
    )(x1, x2, batch3d, wf1, bf1, wf2, bf2)


def kernel(x, edge_index, batch, Wl1, bl1, Wr1, Wl2, bl2, Wr2,
           Wfc1, bfc1, Wfc2, bfc2):
    src = edge_index[0]
    dst = edge_index[1]
    zeros = jnp.zeros((ZCH, D), jnp.float32)
    ones = jnp.ones((CH, D), jnp.float32)

    degw = _sc_deg(dst, ones, zeros).reshape(NC, N, D)
    parts1 = _sc_feat(x, src, dst, zeros)
    x1 = _tc_combine(parts1.reshape(NC, N, D), degw, x,
                     Wl1, bl1.reshape(1, D), Wr1)

    parts2 = _sc_feat(x1, src, dst, zeros)
    x2 = _tc_combine(parts2.reshape(NC, N, D), degw, x1,
                     Wl2, bl2.reshape(1, D), Wr2)

    return _pool_mlp(x1, x2, batch.reshape(NPB, PBLK, 1),
                     Wfc1, bfc1.reshape(1, D), Wfc2, bfc2.reshape(1, C))


# speedup vs baseline: 9.9093x; 2.0808x over previous
import jax
import jax.numpy as jnp
from jax import lax
from jax.experimental import pallas as pl
from jax.experimental.pallas import tpu as pltpu
from jax.experimental.pallas import tpu_sc as plsc

N = 10000
E = 320000
D = 128
C = 10
G = 64

NC = 2
NS = 16
NW = NC * NS
EP = E // NW
CH = 80
NCH = EP // CH

WSTR = 624
TAIL = 16
TOFF = NS * WSTR
ZCH = 48
NZC = WSTR // ZCH

BLK = 2000
PBLK = 1000
NPB = N // PBLK


def _stage_zero(z_h, zb_v, acc_s, sid):
    pltpu.sync_copy(z_h, zb_v)
    for k in range(NZC):
        pltpu.sync_copy(zb_v, acc_s.at[pl.ds(sid * WSTR + k * ZCH, ZCH)])

    @pl.when(sid == 0)
    def _():
        pltpu.sync_copy(zb_v.at[pl.ds(0, TAIL)], acc_s.at[pl.ds(TOFF, TAIL)])


def _stage_out(acc_s, zb_v, out_h, cid, sid):
    for k in range(NZC):
        r0 = sid * WSTR + k * ZCH
        pltpu.sync_copy(acc_s.at[pl.ds(r0, ZCH)], zb_v)
        pltpu.sync_copy(zb_v, out_h.at[pl.ds(cid * N + r0, ZCH)])

    @pl.when(sid == 0)
    def _():
        pltpu.sync_copy(acc_s.at[pl.ds(TOFF, TAIL)], zb_v.at[pl.ds(0, TAIL)])
        pltpu.sync_copy(zb_v.at[pl.ds(0, TAIL)],
                        out_h.at[pl.ds(cid * N + TOFF, TAIL)])


def _copy_idx16(src_ref, off, dst_ref):
    for j in range(CH // 16):
        dst_ref[pl.ds(j * 16, 16)] = src_ref[pl.ds(off + j * 16, 16)]


def _sc_body_deg(dst_h, ones_h, zeros_h, degw_h,
                 acc_s, didall_v, didx_v, ones_v, zb_v):
    cid = lax.axis_index("c")
    sid = lax.axis_index("s")
    wid = cid * NS + sid

    _stage_zero(zeros_h, zb_v, acc_s, sid)
    pltpu.sync_copy(ones_h, ones_v)
    pltpu.sync_copy(dst_h.at[pl.ds(wid * EP, EP)], didall_v)
    plsc.subcore_barrier()

    def _chunk(k, carry):
        _copy_idx16(didall_v, k * CH, didx_v)
        pltpu.sync_copy(ones_v, acc_s.at[didx_v], add=True)
        return carry
    lax.fori_loop(0, NCH, _chunk, 0)

    plsc.subcore_barrier()
    _stage_out(acc_s, zb_v, degw_h, cid, sid)


def _sc_body_feat(x_h, src_h, dst_h, zeros_h, parts_h,
                  acc_s, sidall_v, didall_v, sidx0_v, sidx1_v, didx_v,
                  rows0_v, rows1_v, zb_v, gsem0, gsem1, ssem):
    cid = lax.axis_index("c")
    sid = lax.axis_index("s")
    wid = cid * NS + sid

    _stage_zero(zeros_h, zb_v, acc_s, sid)
    pltpu.sync_copy(src_h.at[pl.ds(wid * EP, EP)], sidall_v)
    pltpu.sync_copy(dst_h.at[pl.ds(wid * EP, EP)], didall_v)
    plsc.subcore_barrier()

    bufs = ((sidx0_v, rows0_v, gsem0), (sidx1_v, rows1_v, gsem1))

    for b in range(2):
        sidx_v, rows_v, gsem = bufs[b]
        _copy_idx16(sidall_v, b * CH, sidx_v)
        pltpu.async_copy(x_h.at[sidx_v], rows_v, gsem)

    def _step(k, b):
        sidx_v, rows_v, gsem = bufs[b]
        _copy_idx16(didall_v, k * CH, didx_v)
        pltpu.make_async_copy(x_h.at[sidx_v], rows_v, gsem).wait()
        pltpu.async_copy(rows_v, acc_s.at[didx_v], ssem, add=True).wait()

        @pl.when(k < NCH - 2)
        def _():
            _copy_idx16(sidall_v, (k + 2) * CH, sidx_v)
            pltpu.async_copy(x_h.at[sidx_v], rows_v, gsem)

    def _pair(g, carry):
        _step(2 * g, 0)
        _step(2 * g + 1, 1)
        return carry
    lax.fori_loop(0, NCH // 2, _pair, 0)
    _step(NCH - 1, 0)

    plsc.subcore_barrier()
    _stage_out(acc_s, zb_v, parts_h, cid, sid)


def _sc_deg(dst, ones, zeros):
    mesh = plsc.VectorSubcoreMesh(core_axis_name="c", subcore_axis_name="s")
    f = pl.kernel(
        _sc_body_deg,
        out_type=jax.ShapeDtypeStruct((NC * N, D), jnp.float32),
        mesh=mesh,
        scratch_types=[
            pltpu.VMEM_SHARED((N, D), jnp.float32),
            pltpu.VMEM((EP,), jnp.int32),
            pltpu.VMEM((CH,), jnp.int32),
            pltpu.VMEM((CH, D), jnp.float32),
            pltpu.VMEM((ZCH, D), jnp.float32),
        ],
    )
    return f(dst, ones, zeros)


def _sc_feat(x, src, dst, zeros):
    mesh = plsc.VectorSubcoreMesh(core_axis_name="c", subcore_axis_name="s")
    f = pl.kernel(
        _sc_body_feat,
        out_type=jax.ShapeDtypeStruct((NC * N, D), jnp.float32),
        mesh=mesh,
        scratch_types=[
            pltpu.VMEM_SHARED((N, D), jnp.float32),
            pltpu.VMEM((EP,), jnp.int32),
            pltpu.VMEM((EP,), jnp.int32),
            pltpu.VMEM((CH,), jnp.int32),
            pltpu.VMEM((CH,), jnp.int32),
            pltpu.VMEM((CH,), jnp.int32),
            pltpu.VMEM((CH, D), jnp.float32),
            pltpu.VMEM((CH, D), jnp.float32),
            pltpu.VMEM((ZCH, D), jnp.float32),
            pltpu.SemaphoreType.DMA,
            pltpu.SemaphoreType.DMA,
            pltpu.SemaphoreType.DMA,
        ],
    )
    return f(x, src, dst, zeros)


def _combine_body(parts_ref, degw_ref, x_ref, wl_ref, bl_ref, wr_ref, out_ref):
    s = parts_ref[0] + parts_ref[1]
    deg = degw_ref[0] + degw_ref[1]
    agg = s / jnp.maximum(deg, 1.0)
    out_ref[...] = (
        lax.dot_general(agg, wl_ref[...], (((1,), (1,)), ((), ())),
                        preferred_element_type=jnp.float32)
        + lax.dot_general(x_ref[...], wr_ref[...], (((1,), (1,)), ((), ())),
                          preferred_element_type=jnp.float32)
        + bl_ref[...]
    )


def _tc_combine(parts, degw, x, wl, bl, wr):
    grid = (N // BLK,)
    return pl.pallas_call(
        _combine_body,
        grid=grid,
        in_specs=[
            pl.BlockSpec((NC, BLK, D), lambda i: (0, i, 0)),
            pl.BlockSpec((NC, BLK, D), lambda i: (0, i, 0)),
            pl.BlockSpec((BLK, D), lambda i: (i, 0)),
            pl.BlockSpec((D, D), lambda i: (0, 0)),
            pl.BlockSpec((1, D), lambda i: (0, 0)),
            pl.BlockSpec((D, D), lambda i: (0, 0)),
        ],
        out_specs=pl.BlockSpec((BLK, D), lambda i: (i, 0)),
        out_shape=jax.ShapeDtypeStruct((N, D), jnp.float32),
    )(parts, degw, x, wl, bl, wr)


def _pool_body(x1_ref, x2_ref, b_ref, wf1_ref, bf1_ref, wf2_ref, bf2_ref,
               out_ref, acc_ref):
    i = pl.program_id(0)

    @pl.when(i == 0)
    def _():
        acc_ref[...] = jnp.full((G, 2 * D), -jnp.inf, jnp.float32)

    bv = b_ref[0]
    b_lo = jnp.min(bv)
    b_hi = jnp.max(bv)
    h1 = x1_ref[...]
    h2 = x2_ref[...]
    neg = jnp.float32(-jnp.inf)

    def _seg(b, carry):
        m = bv == b
        m1 = jnp.max(jnp.where(m, h1, neg), axis=0)
        m2 = jnp.max(jnp.where(m, h2, neg), axis=0)
        row = jnp.concatenate([m1, m2])[None, :]
        acc_ref[pl.ds(b, 1), :] = jnp.maximum(acc_ref[pl.ds(b, 1), :], row)
        return carry
    lax.fori_loop(b_lo, b_hi + 1, _seg, 0)

    @pl.when(i == NPB - 1)
    def _():
        pooled = acc_ref[...]
        z = lax.dot_general(pooled, wf1_ref[...], (((1,), (1,)), ((), ())),
                            preferred_element_type=jnp.float32) + bf1_ref[...]
        z = jnp.maximum(z, 0.0)
        logits = lax.dot_general(z, wf2_ref[...], (((1,), (1,)), ((), ())),
                                 preferred_element_type=jnp.float32) + bf2_ref[...]
        mx = jnp.max(logits, axis=-1, keepdims=True)
        sh = logits - mx
        lse = jnp.log(jnp.sum(jnp.exp(sh), axis=-1, keepdims=True))
        out_ref[...] = sh - lse


def _pool_mlp(x1, x2, batch3d, wf1, bf1, wf2, bf2):
    grid = (NPB,)
    return pl.pallas_call(
        _pool_body,
        grid=grid,
        in_specs=[
            pl.BlockSpec((PBLK, D), lambda i: (i, 0)),
            pl.BlockSpec((PBLK, D), lambda i: (i, 0)),
            pl.BlockSpec((1, PBLK, 1), lambda i: (i, 0, 0)),
            pl.BlockSpec((D, 2 * D), lambda i: (0, 0)),
            pl.BlockSpec((1, D), lambda i: (0, 0)),
            pl.BlockSpec((C, D), lambda i: (0, 0)),
            pl.BlockSpec((1, C), lambda i: (0, 0)),
        ],
        out_specs=pl.BlockSpec((G, C), lambda i: (0, 0)),
        out_shape=jax.ShapeDtypeStruct((G, C), jnp.float32),
        scratch_shapes=[pltpu.VMEM((G, 2 * D), jnp.float32)],
    )(x1, x2, batch3d, wf1, bf1, wf2, bf2)


def kernel(x, edge_index, batch, Wl1, bl1, Wr1, Wl2, bl2, Wr2,
           Wfc1, bfc1, Wfc2, bfc2):
    src = edge_index[0]
    dst = edge_index[1]
    zeros = jnp.zeros((ZCH, D), jnp.float32)
    ones = jnp.ones((CH, D), jnp.float32)

    degw = _sc_deg(dst, ones, zeros).reshape(NC, N, D)
    parts1 = _sc_feat(x, src, dst, zeros)
    x1 = _tc_combine(parts1.reshape(NC, N, D), degw, x,
                     Wl1, bl1.reshape(1, D), Wr1)

    parts2 = _sc_feat(x1, src, dst, zeros)
    x2 = _tc_combine(parts2.reshape(NC, N, D), degw, x1,
                     Wl2, bl2.reshape(1, D), Wr2)

    return _pool_mlp(x1, x2, batch.reshape(NPB, PBLK, 1),
                     Wfc1, bfc1.reshape(1, D), Wfc2, bfc2.reshape(1, C))


# speedup vs baseline: 9.9337x; 1.0025x over previous
import jax
import jax.numpy as jnp
from jax import lax
from jax.experimental import pallas as pl
from jax.experimental.pallas import tpu as pltpu
from jax.experimental.pallas import tpu_sc as plsc

N = 10000
E = 320000
D = 128
C = 10
G = 64

NC = 2
NS = 16
NW = NC * NS
EP = E // NW
CH = 80
NCH = EP // CH

WSTR = 624
TAIL = 16
TOFF = NS * WSTR
ZCH = 48
NZC = WSTR // ZCH

BLK = 2000
NB = N // BLK
DW = 16


def _stage_zero(z_h, zb_v, acc_s, sid):
    pltpu.sync_copy(z_h, zb_v)
    for k in range(NZC):
        pltpu.sync_copy(zb_v, acc_s.at[pl.ds(sid * WSTR + k * ZCH, ZCH)])

    @pl.when(sid == 0)
    def _():
        pltpu.sync_copy(zb_v.at[pl.ds(0, TAIL)], acc_s.at[pl.ds(TOFF, TAIL)])


def _stage_out(acc_s, zb_v, out_h, cid, sid):
    for k in range(NZC):
        r0 = sid * WSTR + k * ZCH
        pltpu.sync_copy(acc_s.at[pl.ds(r0, ZCH)], zb_v)
        pltpu.sync_copy(zb_v, out_h.at[pl.ds(cid * N + r0, ZCH)])

    @pl.when(sid == 0)
    def _():
        pltpu.sync_copy(acc_s.at[pl.ds(TOFF, TAIL)], zb_v.at[pl.ds(0, TAIL)])
        pltpu.sync_copy(zb_v.at[pl.ds(0, TAIL)],
                        out_h.at[pl.ds(cid * N + TOFF, TAIL)])


def _copy_idx16(src_ref, off, dst_ref):
    for j in range(CH // 16):
        dst_ref[pl.ds(j * 16, 16)] = src_ref[pl.ds(off + j * 16, 16)]


def _sc_body_deg(dst_h, ones_h, zeros_h, degw_h,
                 acc_s, didall_v, didx_v, ones_v, zb_v):
    cid = lax.axis_index("c")
    sid = lax.axis_index("s")
    wid = cid * NS + sid

    _stage_zero(zeros_h, zb_v, acc_s, sid)
    pltpu.sync_copy(ones_h, ones_v)
    pltpu.sync_copy(dst_h.at[pl.ds(wid * EP, EP)], didall_v)
    plsc.subcore_barrier()

    def _chunk(k, carry):
        _copy_idx16(didall_v, k * CH, didx_v)
        pltpu.sync_copy(ones_v, acc_s.at[didx_v], add=True)
        return carry
    lax.fori_loop(0, NCH, _chunk, 0)

    plsc.subcore_barrier()
    _stage_out(acc_s, zb_v, degw_h, cid, sid)


def _sc_body_feat(x_h, src_h, dst_h, zeros_h, parts_h,
                  acc_s, sidall_v, didall_v, sidx0_v, sidx1_v, didx_v,
                  rows0_v, rows1_v, zb_v, gsem0, gsem1, ssem):
    cid = lax.axis_index("c")
    sid = lax.axis_index("s")
    wid = cid * NS + sid

    _stage_zero(zeros_h, zb_v, acc_s, sid)
    pltpu.sync_copy(src_h.at[pl.ds(wid * EP, EP)], sidall_v)
    pltpu.sync_copy(dst_h.at[pl.ds(wid * EP, EP)], didall_v)
    plsc.subcore_barrier()

    bufs = ((sidx0_v, rows0_v, gsem0), (sidx1_v, rows1_v, gsem1))

    for b in range(2):
        sidx_v, rows_v, gsem = bufs[b]
        _copy_idx16(sidall_v, b * CH, sidx_v)
        pltpu.async_copy(x_h.at[sidx_v], rows_v, gsem)

    def _step(k, b):
        sidx_v, rows_v, gsem = bufs[b]
        _copy_idx16(didall_v, k * CH, didx_v)
        pltpu.make_async_copy(x_h.at[sidx_v], rows_v, gsem).wait()
        pltpu.async_copy(rows_v, acc_s.at[didx_v], ssem, add=True).wait()

        @pl.when(k < NCH - 2)
        def _():
            _copy_idx16(sidall_v, (k + 2) * CH, sidx_v)
            pltpu.async_copy(x_h.at[sidx_v], rows_v, gsem)

    def _pair(g, carry):
        _step(2 * g, 0)
        _step(2 * g + 1, 1)
        return carry
    lax.fori_loop(0, NCH // 2, _pair, 0)
    _step(NCH - 1, 0)

    plsc.subcore_barrier()
    _stage_out(acc_s, zb_v, parts_h, cid, sid)


def _sc_deg(dst, ones, zeros):
    mesh = plsc.VectorSubcoreMesh(core_axis_name="c", subcore_axis_name="s")
    f = pl.kernel(
        _sc_body_deg,
        out_type=jax.ShapeDtypeStruct((NC * N, D), jnp.float32),
        mesh=mesh,
        scratch_types=[
            pltpu.VMEM_SHARED((N, D), jnp.float32),
            pltpu.VMEM((EP,), jnp.int32),
            pltpu.VMEM((CH,), jnp.int32),
            pltpu.VMEM((CH, D), jnp.float32),
            pltpu.VMEM((ZCH, D), jnp.float32),
        ],
    )
    return f(dst, ones, zeros)


def _sc_feat(x, src, dst, zeros):
    mesh = plsc.VectorSubcoreMesh(core_axis_name="c", subcore_axis_name="s")
    f = pl.kernel(
        _sc_body_feat,
        out_type=jax.ShapeDtypeStruct((NC * N, D), jnp.float32),
        mesh=mesh,
        scratch_types=[
            pltpu.VMEM_SHARED((N, D), jnp.float32),
            pltpu.VMEM((EP,), jnp.int32),
            pltpu.VMEM((EP,), jnp.int32),
            pltpu.VMEM((CH,), jnp.int32),
            pltpu.VMEM((CH,), jnp.int32),
            pltpu.VMEM((CH,), jnp.int32),
            pltpu.VMEM((CH, D), jnp.float32),
            pltpu.VMEM((CH, D), jnp.float32),
            pltpu.VMEM((ZCH, D), jnp.float32),
            pltpu.SemaphoreType.DMA,
            pltpu.SemaphoreType.DMA,
            pltpu.SemaphoreType.DMA,
        ],
    )
    return f(x, src, dst, zeros)


def _sage_out(parts_ref, degw_ref, x_ref, wl_ref, bl_ref, wr_ref):
    s = parts_ref[0] + parts_ref[1]
    deg = (degw_ref[0] + degw_ref[1])[:, 0:1]
    agg = s / jnp.maximum(deg, 1.0)
    return (
        lax.dot_general(agg, wl_ref[...], (((1,), (1,)), ((), ())),
                        preferred_element_type=jnp.float32)
        + lax.dot_general(x_ref[...], wr_ref[...], (((1,), (1,)), ((), ())),
                          preferred_element_type=jnp.float32)
        + bl_ref[...]
    )


def _combine_body(parts_ref, degw_ref, x_ref, wl_ref, bl_ref, wr_ref, out_ref):
    out_ref[...] = _sage_out(parts_ref, degw_ref, x_ref, wl_ref, bl_ref, wr_ref)


def _tc_combine(parts, degw, x, wl, bl, wr):
    grid = (NB,)
    return pl.pallas_call(
        _combine_body,
        grid=grid,
        in_specs=[
            pl.BlockSpec((NC, BLK, D), lambda i: (0, i, 0)),
            pl.BlockSpec((NC, BLK, D), lambda i: (0, i, 0)),
            pl.BlockSpec((BLK, D), lambda i: (i, 0)),
            pl.BlockSpec((D, D), lambda i: (0, 0)),
            pl.BlockSpec((1, D), lambda i: (0, 0)),
            pl.BlockSpec((D, D), lambda i: (0, 0)),
        ],
        out_specs=pl.BlockSpec((BLK, D), lambda i: (i, 0)),
        out_shape=jax.ShapeDtypeStruct((N, D), jnp.float32),
    )(parts, degw, x, wl, bl, wr)


def _combine_pool_body(parts_ref, degw_ref, x1_ref, wl_ref, bl_ref, wr_ref,
                       b_ref, wf1_ref, bf1_ref, wf2_ref, bf2_ref,
                       out_ref, acc_ref):
    i = pl.program_id(0)

    @pl.when(i == 0)
    def _():
        acc_ref[...] = jnp.full((G, 2 * D), -jnp.inf, jnp.float32)

    h2 = _sage_out(parts_ref, degw_ref, x1_ref, wl_ref, bl_ref, wr_ref)
    h1 = x1_ref[...]
    bv = b_ref[0]
    b_lo = jnp.min(bv)
    b_hi = jnp.max(bv)
    neg = jnp.float32(-jnp.inf)

    def _seg(b, carry):
        m = bv == b
        m1 = jnp.max(jnp.where(m, h1, neg), axis=0)
        m2 = jnp.max(jnp.where(m, h2, neg), axis=0)
        row = jnp.concatenate([m1, m2])[None, :]
        acc_ref[pl.ds(b, 1), :] = jnp.maximum(acc_ref[pl.ds(b, 1), :], row)
        return carry
    lax.fori_loop(b_lo, b_hi + 1, _seg, 0)

    @pl.when(i == NB - 1)
    def _():
        pooled = acc_ref[...]
        z = lax.dot_general(pooled, wf1_ref[...], (((1,), (1,)), ((), ())),
                            preferred_element_type=jnp.float32) + bf1_ref[...]
        z = jnp.maximum(z, 0.0)
        logits = lax.dot_general(z, wf2_ref[...], (((1,), (1,)), ((), ())),
                                 preferred_element_type=jnp.float32) + bf2_ref[...]
        mx = jnp.max(logits, axis=-1, keepdims=True)
        sh = logits - mx
        lse = jnp.log(jnp.sum(jnp.exp(sh), axis=-1, keepdims=True))
        out_ref[...] = sh - lse


def _combine_pool(parts, degw, x1, wl, bl, wr, batch3d, wf1, bf1, wf2, bf2):
    grid = (NB,)
    return pl.pallas_call(
        _combine_pool_body,
        grid=grid,
        in_specs=[
            pl.BlockSpec((NC, BLK, D), lambda i: (0, i, 0)),
            pl.BlockSpec((NC, BLK, D), lambda i: (0, i, 0)),
            pl.BlockSpec((BLK, D), lambda i: (i, 0)),
            pl.BlockSpec((D, D), lambda i: (0, 0)),
            pl.BlockSpec((1, D), lambda i: (0, 0)),
            pl.BlockSpec((D, D), lambda i: (0, 0)),
            pl.BlockSpec((1, BLK, 1), lambda i: (i, 0, 0)),
            pl.BlockSpec((D, 2 * D), lambda i: (0, 0)),
            pl.BlockSpec((1, D), lambda i: (0, 0)),
            pl.BlockSpec((C, D), lambda i: (0, 0)),
            pl.BlockSpec((1, C), lambda i: (0, 0)),
        ],
        out_specs=pl.BlockSpec((G, C), lambda i: (0, 0)),
        out_shape=jax.ShapeDtypeStruct((G, C), jnp.float32),
        scratch_shapes=[pltpu.VMEM((G, 2 * D), jnp.float32)],
    )(parts, degw, x1, wl, bl, wr, batch3d, wf1, bf1, wf2, bf2)


def kernel(x, edge_index, batch, Wl1, bl1, Wr1, Wl2, bl2, Wr2,
           Wfc1, bfc1, Wfc2, bfc2):
    src = edge_index[0]
    dst = edge_index[1]
    zeros = jnp.zeros((ZCH, D), jnp.float32)
    ones = jnp.ones((CH, D), jnp.float32)

    degw = _sc_deg(dst, ones, zeros).reshape(NC, N, D)
    parts1 = _sc_feat(x, src, dst, zeros)
    x1 = _tc_combine(parts1.reshape(NC, N, D), degw, x,
                     Wl1, bl1.reshape(1, D), Wr1)

    parts2 = _sc_feat(x1, src, dst, zeros)
    return _combine_pool(parts2.reshape(NC, N, D), degw, x1,
                         Wl2, bl2.reshape(1, D), Wr2,
                         batch.reshape(NB, BLK, 1),
                         Wfc1, bfc1.reshape(1, D), Wfc2, bfc2.reshape(1, C))


# speedup vs baseline: 11.0381x; 1.1112x over previous
import jax
import jax.numpy as jnp
from jax import lax
from jax.experimental import pallas as pl
from jax.experimental.pallas import tpu as pltpu
from jax.experimental.pallas import tpu_sc as plsc

N = 10000
E = 320000
D = 128
C = 10
G = 64

NC = 2
NS = 16
NW = NC * NS
EP = E // NW
CH = 80
NCH = EP // CH

WSTR = 624
TAIL = 16
TOFF = NS * WSTR
ZCH = 48
NZC = WSTR // ZCH

BLK = 2000
NB = N // BLK
DW = 16


def _stage_zero(z_h, zb_v, acc_s, sid):
    pltpu.sync_copy(z_h, zb_v)
    for k in range(NZC):
        pltpu.sync_copy(zb_v, acc_s.at[pl.ds(sid * WSTR + k * ZCH, ZCH)])

    @pl.when(sid == 0)
    def _():
        pltpu.sync_copy(zb_v.at[pl.ds(0, TAIL)], acc_s.at[pl.ds(TOFF, TAIL)])


def _stage_out(acc_s, zb_v, out_h, cid, sid):
    for k in range(NZC):
        r0 = sid * WSTR + k * ZCH
        pltpu.sync_copy(acc_s.at[pl.ds(r0, ZCH)], zb_v)
        pltpu.sync_copy(zb_v, out_h.at[pl.ds(cid * N + r0, ZCH)])

    @pl.when(sid == 0)
    def _():
        pltpu.sync_copy(acc_s.at[pl.ds(TOFF, TAIL)], zb_v.at[pl.ds(0, TAIL)])
        pltpu.sync_copy(zb_v.at[pl.ds(0, TAIL)],
                        out_h.at[pl.ds(cid * N + TOFF, TAIL)])


def _copy_idx16(src_ref, off, dst_ref):
    for j in range(CH // 16):
        dst_ref[pl.ds(j * 16, 16)] = src_ref[pl.ds(off + j * 16, 16)]


def _sc_body_deg(dst_h, ones_h, zeros_h, degw_h,
                 acc_s, didall_v, didx_v, didx1_v, ones_v, zb_v,
                 ssem0, ssem1):
    cid = lax.axis_index("c")
    sid = lax.axis_index("s")
    wid = cid * NS + sid

    _stage_zero(zeros_h, zb_v, acc_s, sid)
    pltpu.sync_copy(ones_h, ones_v)
    pltpu.sync_copy(dst_h.at[pl.ds(wid * EP, EP)], didall_v)
    plsc.subcore_barrier()

    didxs = (didx_v, didx1_v)
    ssems = (ssem0, ssem1)

    def _dstep(k, b):
        _copy_idx16(didall_v, k * CH, didxs[b])

        @pl.when(k >= 1)
        def _():
            pltpu.make_async_copy(ones_v, acc_s.at[didxs[1 - b]],
                                  ssems[1 - b]).wait()
        pltpu.async_copy(ones_v, acc_s.at[didxs[b]], ssems[b], add=True)

    def _dpair(g, carry):
        _dstep(2 * g, 0)
        _dstep(2 * g + 1, 1)
        return carry
    lax.fori_loop(0, NCH // 2, _dpair, 0)
    _dstep(NCH - 1, 0)
    pltpu.make_async_copy(ones_v, acc_s.at[didxs[0]], ssems[0]).wait()

    plsc.subcore_barrier()
    _stage_out(acc_s, zb_v, degw_h, cid, sid)


def _sc_body_feat(x_h, src_h, dst_h, zeros_h, parts_h,
                  acc_s, sidall_v, sidx0_v, sidx1_v,
                  didx0_v, didx1_v, didx2_v, rows0_v, rows1_v, rows2_v,
                  gsem0, gsem1, ssem0, ssem1, isem0, isem1):
    cid = lax.axis_index("c")
    sid = lax.axis_index("s")
    wid = cid * NS + sid

    zb_v = rows0_v.at[pl.ds(0, ZCH)]
    _stage_zero(zeros_h, zb_v, acc_s, sid)
    pltpu.sync_copy(src_h.at[pl.ds(wid * EP, EP)], sidall_v)
    plsc.subcore_barrier()

    rows = (rows0_v, rows1_v, rows2_v)
    sidxs = (sidx0_v, sidx1_v)
    didxs = (didx0_v, didx1_v, didx2_v)
    gsems = (gsem0, gsem1)
    ssems = (ssem0, ssem1)
    isems = (isem0, isem1)
    base = wid * EP

    for b in range(2):
        pltpu.async_copy(dst_h.at[pl.ds(base + b * CH, CH)], didxs[b],
                         isems[b])
        _copy_idx16(sidall_v, b * CH, sidxs[b])
        pltpu.async_copy(x_h.at[sidxs[b]], rows[b], gsems[b])

    def _step(k, b3, b2):
        pltpu.make_async_copy(dst_h.at[pl.ds(0, CH)], didxs[b3],
                              isems[b2]).wait()
        pltpu.make_async_copy(x_h.at[sidxs[b2]], rows[b3], gsems[b2]).wait()
        pltpu.async_copy(rows[b3], acc_s.at[didxs[b3]], ssems[b2], add=True)

        @pl.when(k >= 1)
        def _():
            pltpu.make_async_copy(rows[0], acc_s.at[didxs[0]],
                                  ssems[1 - b2]).wait()

        @pl.when(k < NCH - 2)
        def _():
            _copy_idx16(sidall_v, (k + 2) * CH, sidxs[b2])
            pltpu.async_copy(x_h.at[sidxs[b2]], rows[(b3 + 2) % 3],
                             gsems[b2])
            pltpu.async_copy(dst_h.at[pl.ds(base + (k + 2) * CH, CH)],
                             didxs[(b3 + 2) % 3], isems[b2])

    def _six(g, carry):
        for j in range(6):
            _step(6 * g + j, j % 3, j % 2)
        return carry
    lax.fori_loop(0, NCH // 6, _six, 0)
    for k in range(NCH - NCH % 6, NCH):
        _step(k, k % 3, k % 2)
    pltpu.make_async_copy(rows[0], acc_s.at[didxs[0]],
                          ssems[(NCH - 1) % 2]).wait()

    plsc.subcore_barrier()
    _stage_out(acc_s, zb_v, parts_h, cid, sid)


def _sc_deg(dst, ones, zeros):
    mesh = plsc.VectorSubcoreMesh(core_axis_name="c", subcore_axis_name="s")
    f = pl.kernel(
        _sc_body_deg,
        out_type=jax.ShapeDtypeStruct((NC * N, D), jnp.float32),
        mesh=mesh,
        scratch_types=[
            pltpu.VMEM_SHARED((N, D), jnp.float32),
            pltpu.VMEM((EP,), jnp.int32),
            pltpu.VMEM((CH,), jnp.int32),
            pltpu.VMEM((CH,), jnp.int32),
            pltpu.VMEM((CH, D), jnp.float32),
            pltpu.VMEM((ZCH, D), jnp.float32),
            pltpu.SemaphoreType.DMA,
            pltpu.SemaphoreType.DMA,
        ],
    )
    return f(dst, ones, zeros)


def _sc_feat(x, src, dst, zeros):
    mesh = plsc.VectorSubcoreMesh(core_axis_name="c", subcore_axis_name="s")
    f = pl.kernel(
        _sc_body_feat,
        out_type=jax.ShapeDtypeStruct((NC * N, D), jnp.float32),
        mesh=mesh,
        scratch_types=[
            pltpu.VMEM_SHARED((N, D), jnp.float32),
            pltpu.VMEM((EP,), jnp.int32),
            pltpu.VMEM((CH,), jnp.int32),
            pltpu.VMEM((CH,), jnp.int32),
            pltpu.VMEM((CH,), jnp.int32),
            pltpu.VMEM((CH,), jnp.int32),
            pltpu.VMEM((CH,), jnp.int32),
            pltpu.VMEM((CH, D), jnp.float32),
            pltpu.VMEM((CH, D), jnp.float32),
            pltpu.VMEM((CH, D), jnp.float32),
            pltpu.SemaphoreType.DMA,
            pltpu.SemaphoreType.DMA,
            pltpu.SemaphoreType.DMA,
            pltpu.SemaphoreType.DMA,
            pltpu.SemaphoreType.DMA,
            pltpu.SemaphoreType.DMA,
        ],
    )
    return f(x, src, dst, zeros)


def _sage_out(parts_ref, degw_ref, x_ref, wl_ref, bl_ref, wr_ref):
    s = parts_ref[0] + parts_ref[1]
    deg = (degw_ref[0] + degw_ref[1])[:, 0:1]
    agg = s / jnp.maximum(deg, 1.0)
    return (
        lax.dot_general(agg, wl_ref[...], (((1,), (1,)), ((), ())),
                        preferred_element_type=jnp.float32)
        + lax.dot_general(x_ref[...], wr_ref[...], (((1,), (1,)), ((), ())),
                          preferred_element_type=jnp.float32)
        + bl_ref[...]
    )


def _combine_body(parts_ref, degw_ref, x_ref, wl_ref, bl_ref, wr_ref, out_ref):
    out_ref[...] = _sage_out(parts_ref, degw_ref, x_ref, wl_ref, bl_ref, wr_ref)


def _tc_combine(parts, degw, x, wl, bl, wr):
    grid = (NB,)
    return pl.pallas_call(
        _combine_body,
        grid=grid,
        in_specs=[
            pl.BlockSpec((NC, BLK, D), lambda i: (0, i, 0)),
            pl.BlockSpec((NC, BLK, D), lambda i: (0, i, 0)),
            pl.BlockSpec((BLK, D), lambda i: (i, 0)),
            pl.BlockSpec((D, D), lambda i: (0, 0)),
            pl.BlockSpec((1, D), lambda i: (0, 0)),
            pl.BlockSpec((D, D), lambda i: (0, 0)),
        ],
        out_specs=pl.BlockSpec((BLK, D), lambda i: (i, 0)),
        out_shape=jax.ShapeDtypeStruct((N, D), jnp.float32),
    )(parts, degw, x, wl, bl, wr)


def _combine_pool_body(parts_ref, degw_ref, x1_ref, wl_ref, bl_ref, wr_ref,
                       b_ref, wf1_ref, bf1_ref, wf2_ref, bf2_ref,
                       out_ref, acc_ref):
    i = pl.program_id(0)

    @pl.when(i == 0)
    def _():
        acc_ref[...] = jnp.full((G, 2 * D), -jnp.inf, jnp.float32)

    h2 = _sage_out(parts_ref, degw_ref, x1_ref, wl_ref, bl_ref, wr_ref)
    h1 = x1_ref[...]
    bv = b_ref[0]
    b_lo = jnp.min(bv)
    b_hi = jnp.max(bv)
    neg = jnp.float32(-jnp.inf)

    def _seg(b, carry):
        m = bv == b
        m1 = jnp.max(jnp.where(m, h1, neg), axis=0)
        m2 = jnp.max(jnp.where(m, h2, neg), axis=0)
        row = jnp.concatenate([m1, m2])[None, :]
        acc_ref[pl.ds(b, 1), :] = jnp.maximum(acc_ref[pl.ds(b, 1), :], row)
        return carry
    lax.fori_loop(b_lo, b_hi + 1, _seg, 0)

    @pl.when(i == NB - 1)
    def _():
        pooled = acc_ref[...]
        z = lax.dot_general(pooled, wf1_ref[...], (((1,), (1,)), ((), ())),
                            preferred_element_type=jnp.float32) + bf1_ref[...]
        z = jnp.maximum(z, 0.0)
        logits = lax.dot_general(z, wf2_ref[...], (((1,), (1,)), ((), ())),
                                 preferred_element_type=jnp.float32) + bf2_ref[...]
        mx = jnp.max(logits, axis=-1, keepdims=True)
        sh = logits - mx
        lse = jnp.log(jnp.sum(jnp.exp(sh), axis=-1, keepdims=True))
        out_ref[...] = sh - lse


def _combine_pool(parts, degw, x1, wl, bl, wr, batch3d, wf1, bf1, wf2, bf2):
    grid = (NB,)
    return pl.pallas_call(
        _combine_pool_body,
        grid=grid,
        in_specs=[
            pl.BlockSpec((NC, BLK, D), lambda i: (0, i, 0)),
            pl.BlockSpec((NC, BLK, D), lambda i: (0, i, 0)),
            pl.BlockSpec((BLK, D), lambda i: (i, 0)),
            pl.BlockSpec((D, D), lambda i: (0, 0)),
            pl.BlockSpec((1, D), lambda i: (0, 0)),
            pl.BlockSpec((D, D), lambda i: (0, 0)),
            pl.BlockSpec((1, BLK, 1), lambda i: (i, 0, 0)),
            pl.BlockSpec((D, 2 * D), lambda i: (0, 0)),
            pl.BlockSpec((1, D), lambda i: (0, 0)),
            pl.BlockSpec((C, D), lambda i: (0, 0)),
            pl.BlockSpec((1, C), lambda i: (0, 0)),
        ],
        out_specs=pl.BlockSpec((G, C), lambda i: (0, 0)),
        out_shape=jax.ShapeDtypeStruct((G, C), jnp.float32),
        scratch_shapes=[pltpu.VMEM((G, 2 * D), jnp.float32)],
    )(parts, degw, x1, wl, bl, wr, batch3d, wf1, bf1, wf2, bf2)


def kernel(x, edge_index, batch, Wl1, bl1, Wr1, Wl2, bl2, Wr2,
           Wfc1, bfc1, Wfc2, bfc2):
    src = edge_index[0]
    dst = edge_index[1]
    zeros = jnp.zeros((ZCH, D), jnp.float32)
    ones = jnp.ones((CH, D), jnp.float32)

    degw = _sc_deg(dst, ones, zeros).reshape(NC, N, D)
    parts1 = _sc_feat(x, src, dst, zeros)
    x1 = _tc_combine(parts1.reshape(NC, N, D), degw, x,
                     Wl1, bl1.reshape(1, D), Wr1)

    parts2 = _sc_feat(x1, src, dst, zeros)
    return _combine_pool(parts2.reshape(NC, N, D), degw, x1,
                         Wl2, bl2.reshape(1, D), Wr2,
                         batch.reshape(NB, BLK, 1),
                         Wfc1, bfc1.reshape(1, D), Wfc2, bfc2.reshape(1, C))


# speedup vs baseline: 11.2658x; 1.0206x over previous
import functools

import jax
import jax.numpy as jnp
from jax import lax
from jax.experimental import pallas as pl
from jax.experimental.pallas import tpu as pltpu
from jax.experimental.pallas import tpu_sc as plsc

N = 10000
E = 320000
D = 128
C = 10
G = 64

NC = 2
NS = 16
NW = NC * NS
EP = E // NW
CH = 80
NCH = EP // CH

WSTR = 624
TAIL = 16
TOFF = NS * WSTR
ZCH = 48
NZC = WSTR // ZCH

BLK = 2000
NB = N // BLK
DW = 16


def _stage_zero(z_h, zb_v, acc_s, sid):
    pltpu.sync_copy(z_h, zb_v)
    for k in range(NZC):
        pltpu.sync_copy(zb_v, acc_s.at[pl.ds(sid * WSTR + k * ZCH, ZCH)])

    @pl.when(sid == 0)
    def _():
        pltpu.sync_copy(zb_v.at[pl.ds(0, TAIL)], acc_s.at[pl.ds(TOFF, TAIL)])


def _stage_out(acc_s, zb_v, out_h, cid, sid):
    for k in range(NZC):
        r0 = sid * WSTR + k * ZCH
        pltpu.sync_copy(acc_s.at[pl.ds(r0, ZCH)], zb_v)
        pltpu.sync_copy(zb_v, out_h.at[pl.ds(cid * N + r0, ZCH)])

    @pl.when(sid == 0)
    def _():
        pltpu.sync_copy(acc_s.at[pl.ds(TOFF, TAIL)], zb_v.at[pl.ds(0, TAIL)])
        pltpu.sync_copy(zb_v.at[pl.ds(0, TAIL)],
                        out_h.at[pl.ds(cid * N + TOFF, TAIL)])


def _sc_body_feat(with_deg, *refs):
    if with_deg:
        (x_h, src_h, dst_h, zeros_h, ones_h, parts_h, degw_h,
         acc_s, sidall_v, didx0_v, didx1_v, didx2_v,
         rows0_v, rows1_v, rows2_v,
         gsem0, gsem1, ssem0, ssem1, isem0, isem1) = refs
    else:
        (x_h, src_h, dst_h, zeros_h, parts_h,
         acc_s, sidall_v, didx0_v, didx1_v, didx2_v,
         rows0_v, rows1_v, rows2_v,
         gsem0, gsem1, ssem0, ssem1, isem0, isem1) = refs
    cid = lax.axis_index("c")
    sid = lax.axis_index("s")
    wid = cid * NS + sid

    zb_v = rows1_v.at[pl.ds(0, ZCH)]
    _stage_zero(zeros_h, zb_v, acc_s, sid)
    pltpu.sync_copy(src_h.at[pl.ds(wid * EP, EP)], sidall_v)
    if with_deg:
        pltpu.sync_copy(ones_h, rows0_v)
    plsc.subcore_barrier()

    didxs = (didx0_v, didx1_v, didx2_v)
    ssems = (ssem0, ssem1)
    isems = (isem0, isem1)
    dbase = wid * EP

    if with_deg:
        for b in range(2):
            pltpu.async_copy(dst_h.at[pl.ds(dbase + b * CH, CH)], didxs[b],
                             isems[b])

        def _dstep(k, b3, b2):
            pltpu.make_async_copy(dst_h.at[pl.ds(0, CH)], didxs[b3],
                                  isems[b2]).wait()
            pltpu.async_copy(rows0_v, acc_s.at[didxs[b3]], ssems[b2],
                             add=True)

            @pl.when(k >= 1)
            def _():
                pltpu.make_async_copy(rows0_v, acc_s.at[didxs[0]],
                                      ssems[1 - b2]).wait()

            @pl.when(k < NCH - 2)
            def _():
                pltpu.async_copy(dst_h.at[pl.ds(dbase + (k + 2) * CH, CH)],
                                 didxs[(b3 + 2) % 3], isems[b2])

        def _dsix(g, carry):
            for j in range(6):
                _dstep(6 * g + j, j % 3, j % 2)
            return carry
        lax.fori_loop(0, NCH // 6, _dsix, 0)
        for k in range(NCH - NCH % 6, NCH):
            _dstep(k, k % 3, k % 2)
        pltpu.make_async_copy(rows0_v, acc_s.at[didxs[0]],
                              ssems[(NCH - 1) % 2]).wait()

        plsc.subcore_barrier()
        _stage_out(acc_s, zb_v, degw_h, cid, sid)
        _stage_zero(zeros_h, zb_v, acc_s, sid)
        plsc.subcore_barrier()

    rows = (rows0_v, rows1_v, rows2_v)
    gsems = (gsem0, gsem1)
    base = wid * EP

    for b in range(2):
        pltpu.async_copy(dst_h.at[pl.ds(base + b * CH, CH)], didxs[b],
                         isems[b])
        pltpu.async_copy(x_h.at[sidall_v.at[pl.ds(b * CH, CH)]],
                         rows[b], gsems[b])

    def _step(k, b3, b2):
        pltpu.make_async_copy(dst_h.at[pl.ds(0, CH)], didxs[b3],
                              isems[b2]).wait()
        pltpu.make_async_copy(x_h.at[sidall_v.at[pl.ds(0, CH)]],
                              rows[b3], gsems[b2]).wait()
        pltpu.async_copy(rows[b3], acc_s.at[didxs[b3]], ssems[b2], add=True)

        @pl.when(k >= 1)
        def _():
            pltpu.make_async_copy(rows[0], acc_s.at[didxs[0]],
                                  ssems[1 - b2]).wait()

        @pl.when(k < NCH - 2)
        def _():
            pltpu.async_copy(x_h.at[sidall_v.at[pl.ds((k + 2) * CH, CH)]],
                             rows[(b3 + 2) % 3], gsems[b2])
            pltpu.async_copy(dst_h.at[pl.ds(base + (k + 2) * CH, CH)],
                             didxs[(b3 + 2) % 3], isems[b2])

    def _six(g, carry):
        for j in range(6):
            _step(6 * g + j, j % 3, j % 2)
        return carry
    lax.fori_loop(0, NCH // 6, _six, 0)
    for k in range(NCH - NCH % 6, NCH):
        _step(k, k % 3, k % 2)
    pltpu.make_async_copy(rows[0], acc_s.at[didxs[0]],
                          ssems[(NCH - 1) % 2]).wait()

    plsc.subcore_barrier()
    _stage_out(acc_s, zb_v, parts_h, cid, sid)


_SC_SCRATCH = [
    pltpu.VMEM_SHARED((N, D), jnp.float32),
    pltpu.VMEM((EP,), jnp.int32),
    pltpu.VMEM((CH,), jnp.int32),
    pltpu.VMEM((CH,), jnp.int32),
    pltpu.VMEM((CH,), jnp.int32),
    pltpu.VMEM((CH, D), jnp.float32),
    pltpu.VMEM((CH, D), jnp.float32),
    pltpu.VMEM((CH, D), jnp.float32),
    pltpu.SemaphoreType.DMA,
    pltpu.SemaphoreType.DMA,
    pltpu.SemaphoreType.DMA,
    pltpu.SemaphoreType.DMA,
    pltpu.SemaphoreType.DMA,
    pltpu.SemaphoreType.DMA,
]


def _sc_feat_deg(x, src, dst, zeros, ones):
    mesh = plsc.VectorSubcoreMesh(core_axis_name="c", subcore_axis_name="s")
    f = pl.kernel(
        functools.partial(_sc_body_feat, True),
        out_type=(jax.ShapeDtypeStruct((NC * N, D), jnp.float32),
                  jax.ShapeDtypeStruct((NC * N, D), jnp.float32)),
        mesh=mesh,
        scratch_types=_SC_SCRATCH,
    )
    return f(x, src, dst, zeros, ones)


def _sc_feat(x, src, dst, zeros):
    mesh = plsc.VectorSubcoreMesh(core_axis_name="c", subcore_axis_name="s")
    f = pl.kernel(
        functools.partial(_sc_body_feat, False),
        out_type=jax.ShapeDtypeStruct((NC * N, D), jnp.float32),
        mesh=mesh,
        scratch_types=_SC_SCRATCH,
    )
    return f(x, src, dst, zeros)


def _sage_out(parts_ref, degw_ref, x_ref, wl_ref, bl_ref, wr_ref):
    s = parts_ref[0] + parts_ref[1]
    deg = (degw_ref[0] + degw_ref[1])[:, 0:1]
    agg = s / jnp.maximum(deg, 1.0)
    return (
        lax.dot_general(agg, wl_ref[...], (((1,), (1,)), ((), ())),
                        preferred_element_type=jnp.float32)
        + lax.dot_general(x_ref[...], wr_ref[...], (((1,), (1,)), ((), ())),
                          preferred_element_type=jnp.float32)
        + bl_ref[...]
    )


def _combine_body(parts_ref, degw_ref, x_ref, wl_ref, bl_ref, wr_ref, out_ref):
    out_ref[...] = _sage_out(parts_ref, degw_ref, x_ref, wl_ref, bl_ref, wr_ref)


def _tc_combine(parts, degw, x, wl, bl, wr):
    grid = (NB,)
    return pl.pallas_call(
        _combine_body,
        grid=grid,
        in_specs=[
            pl.BlockSpec((NC, BLK, D), lambda i: (0, i, 0)),
            pl.BlockSpec((NC, BLK, D), lambda i: (0, i, 0)),
            pl.BlockSpec((BLK, D), lambda i: (i, 0)),
            pl.BlockSpec((D, D), lambda i: (0, 0)),
            pl.BlockSpec((1, D), lambda i: (0, 0)),
            pl.BlockSpec((D, D), lambda i: (0, 0)),
        ],
        out_specs=pl.BlockSpec((BLK, D), lambda i: (i, 0)),
        out_shape=jax.ShapeDtypeStruct((N, D), jnp.float32),
    )(parts, degw, x, wl, bl, wr)


def _combine_pool_body(parts_ref, degw_ref, x1_ref, wl_ref, bl_ref, wr_ref,
                       b_ref, wf1_ref, bf1_ref, wf2_ref, bf2_ref,
                       out_ref, acc_ref):
    i = pl.program_id(0)

    @pl.when(i == 0)
    def _():
        acc_ref[...] = jnp.full((G, 2 * D), -jnp.inf, jnp.float32)

    h2 = _sage_out(parts_ref, degw_ref, x1_ref, wl_ref, bl_ref, wr_ref)
    h1 = x1_ref[...]
    bv = b_ref[0]
    b_lo = jnp.min(bv)
    b_hi = jnp.max(bv)
    neg = jnp.float32(-jnp.inf)

    def _seg(b, carry):
        m = bv == b
        m1 = jnp.max(jnp.where(m, h1, neg), axis=0)
        m2 = jnp.max(jnp.where(m, h2, neg), axis=0)
        row = jnp.concatenate([m1, m2])[None, :]
        acc_ref[pl.ds(b, 1), :] = jnp.maximum(acc_ref[pl.ds(b, 1), :], row)
        return carry
    lax.fori_loop(b_lo, b_hi + 1, _seg, 0)

    @pl.when(i == NB - 1)
    def _():
        pooled = acc_ref[...]
        z = lax.dot_general(pooled, wf1_ref[...], (((1,), (1,)), ((), ())),
                            preferred_element_type=jnp.float32) + bf1_ref[...]
        z = jnp.maximum(z, 0.0)
        logits = lax.dot_general(z, wf2_ref[...], (((1,), (1,)), ((), ())),
                                 preferred_element_type=jnp.float32) + bf2_ref[...]
        mx = jnp.max(logits, axis=-1, keepdims=True)
        sh = logits - mx
        lse = jnp.log(jnp.sum(jnp.exp(sh), axis=-1, keepdims=True))
        out_ref[...] = sh - lse


def _combine_pool(parts, degw, x1, wl, bl, wr, batch3d, wf1, bf1, wf2, bf2):
    grid = (NB,)
    return pl.pallas_call(
        _combine_pool_body,
        grid=grid,
        in_specs=[
            pl.BlockSpec((NC, BLK, D), lambda i: (0, i, 0)),
            pl.BlockSpec((NC, BLK, D), lambda i: (0, i, 0)),
            pl.BlockSpec((BLK, D), lambda i: (i, 0)),
            pl.BlockSpec((D, D), lambda i: (0, 0)),
            pl.BlockSpec((1, D), lambda i: (0, 0)),
            pl.BlockSpec((D, D), lambda i: (0, 0)),
            pl.BlockSpec((1, BLK, 1), lambda i: (i, 0, 0)),
            pl.BlockSpec((D, 2 * D), lambda i: (0, 0)),
            pl.BlockSpec((1, D), lambda i: (0, 0)),
            pl.BlockSpec((C, D), lambda i: (0, 0)),
            pl.BlockSpec((1, C), lambda i: (0, 0)),
        ],
        out_specs=pl.BlockSpec((G, C), lambda i: (0, 0)),
        out_shape=jax.ShapeDtypeStruct((G, C), jnp.float32),
        scratch_shapes=[pltpu.VMEM((G, 2 * D), jnp.float32)],
    )(parts, degw, x1, wl, bl, wr, batch3d, wf1, bf1, wf2, bf2)


def kernel(x, edge_index, batch, Wl1, bl1, Wr1, Wl2, bl2, Wr2,
           Wfc1, bfc1, Wfc2, bfc2):
    src = edge_index[0]
    dst = edge_index[1]
    zeros = jnp.zeros((ZCH, D), jnp.float32)
    ones = jnp.ones((CH, D), jnp.float32)

    parts1, degw = _sc_feat_deg(x, src, dst, zeros, ones)
    degw = degw.reshape(NC, N, D)
    x1 = _tc_combine(parts1.reshape(NC, N, D), degw, x,
                     Wl1, bl1.reshape(1, D), Wr1)

    parts2 = _sc_feat(x1, src, dst, zeros)
    return _combine_pool(parts2.reshape(NC, N, D), degw, x1,
                         Wl2, bl2.reshape(1, D), Wr2,
                         batch.reshape(NB, BLK, 1),
                         Wfc1, bfc1.reshape(1, D), Wfc2, bfc2.reshape(1, C))


# speedup vs baseline: 11.6076x; 1.0303x over previous
import functools

import jax
import jax.numpy as jnp
from jax import lax
from jax.experimental import pallas as pl
from jax.experimental.pallas import tpu as pltpu
from jax.experimental.pallas import tpu_sc as plsc

N = 10000
E = 320000
D = 128
C = 10
G = 64

NC = 2
NS = 16
NW = NC * NS
EP = E // NW
CH = 80
NCH = EP // CH

WSTR = 624
TAIL = 16
TOFF = NS * WSTR
ZCH = 48
NZC = WSTR // ZCH

BLK = 2000
NB = N // BLK
DW = 16


def _stage_zero(z_h, zb_v, acc_s, sid, sem):
    pltpu.sync_copy(z_h, zb_v)
    for k in range(NZC):
        pltpu.async_copy(zb_v, acc_s.at[pl.ds(sid * WSTR + k * ZCH, ZCH)],
                         sem)
    for k in range(NZC):
        pltpu.make_async_copy(zb_v, acc_s.at[pl.ds(sid * WSTR, ZCH)],
                              sem).wait()

    @pl.when(sid == 0)
    def _():
        pltpu.sync_copy(zb_v.at[pl.ds(0, TAIL)], acc_s.at[pl.ds(TOFF, TAIL)])


def _stage_out(acc_s, zbs, out_h, cid, sid, rsems, wsems):
    for k in range(NZC):
        p = k % 2
        r0 = sid * WSTR + k * ZCH
        if k >= 2:
            pltpu.make_async_copy(zbs[p], out_h.at[pl.ds(cid * N, ZCH)],
                                  wsems[p]).wait()
        pltpu.async_copy(acc_s.at[pl.ds(r0, ZCH)], zbs[p], rsems[p])
        pltpu.make_async_copy(acc_s.at[pl.ds(r0, ZCH)], zbs[p],
                              rsems[p]).wait()
        pltpu.async_copy(zbs[p], out_h.at[pl.ds(cid * N + r0, ZCH)],
                         wsems[p])
    for p in ((NZC - 2) % 2, (NZC - 1) % 2):
        pltpu.make_async_copy(zbs[p], out_h.at[pl.ds(cid * N, ZCH)],
                              wsems[p]).wait()

    @pl.when(sid == 0)
    def _():
        pltpu.sync_copy(acc_s.at[pl.ds(TOFF, TAIL)],
                        zbs[0].at[pl.ds(0, TAIL)])
        pltpu.sync_copy(zbs[0].at[pl.ds(0, TAIL)],
                        out_h.at[pl.ds(cid * N + TOFF, TAIL)])


def _sc_body_feat(with_deg, *refs):
    if with_deg:
        (x_h, src_h, dst_h, zeros_h, ones_h, parts_h, degw_h,
         acc_s, sidall_v, didx0_v, didx1_v, didx2_v,
         rows0_v, rows1_v, rows2_v,
         gsem0, gsem1, ssem0, ssem1, isem0, isem1) = refs
    else:
        (x_h, src_h, dst_h, zeros_h, parts_h,
         acc_s, sidall_v, didx0_v, didx1_v, didx2_v,
         rows0_v, rows1_v, rows2_v,
         gsem0, gsem1, ssem0, ssem1, isem0, isem1) = refs
    cid = lax.axis_index("c")
    sid = lax.axis_index("s")
    wid = cid * NS + sid

    zb_v = rows1_v.at[pl.ds(0, ZCH)]
    zbs = (zb_v, rows2_v.at[pl.ds(0, ZCH)])
    _stage_zero(zeros_h, zb_v, acc_s, sid, isem0)
    pltpu.sync_copy(src_h.at[pl.ds(wid * EP, EP)], sidall_v)
    if with_deg:
        pltpu.sync_copy(ones_h, rows0_v)
    plsc.subcore_barrier()

    didxs = (didx0_v, didx1_v, didx2_v)
    ssems = (ssem0, ssem1)
    isems = (isem0, isem1)
    dbase = wid * EP

    if with_deg:
        for b in range(2):
            pltpu.async_copy(dst_h.at[pl.ds(dbase + b * CH, CH)], didxs[b],
                             isems[b])

        def _dstep(k, b3, b2):
            pltpu.make_async_copy(dst_h.at[pl.ds(0, CH)], didxs[b3],
                                  isems[b2]).wait()
            pltpu.async_copy(rows0_v, acc_s.at[didxs[b3]], ssems[b2],
                             add=True)

            @pl.when(k >= 1)
            def _():
                pltpu.make_async_copy(rows0_v, acc_s.at[didxs[0]],
                                      ssems[1 - b2]).wait()

            @pl.when(k < NCH - 2)
            def _():
                pltpu.async_copy(dst_h.at[pl.ds(dbase + (k + 2) * CH, CH)],
                                 didxs[(b3 + 2) % 3], isems[b2])

        def _dsix(g, carry):
            for j in range(6):
                _dstep(6 * g + j, j % 3, j % 2)
            return carry
        lax.fori_loop(0, NCH // 6, _dsix, 0)
        for k in range(NCH - NCH % 6, NCH):
            _dstep(k, k % 3, k % 2)
        pltpu.make_async_copy(rows0_v, acc_s.at[didxs[0]],
                              ssems[(NCH - 1) % 2]).wait()

        plsc.subcore_barrier()
        _stage_out(acc_s, zbs, degw_h, cid, sid, (gsem0, gsem1),
                   (ssem0, ssem1))
        _stage_zero(zeros_h, zb_v, acc_s, sid, isem0)
        plsc.subcore_barrier()

    rows = (rows0_v, rows1_v, rows2_v)
    gsems = (gsem0, gsem1)
    base = wid * EP

    for b in range(2):
        pltpu.async_copy(dst_h.at[pl.ds(base + b * CH, CH)], didxs[b],
                         isems[b])
        pltpu.async_copy(x_h.at[sidall_v.at[pl.ds(b * CH, CH)]],
                         rows[b], gsems[b])

    def _step(k, b3, b2):
        pltpu.make_async_copy(dst_h.at[pl.ds(0, CH)], didxs[b3],
                              isems[b2]).wait()
        pltpu.make_async_copy(x_h.at[sidall_v.at[pl.ds(0, CH)]],
                              rows[b3], gsems[b2]).wait()
        pltpu.async_copy(rows[b3], acc_s.at[didxs[b3]], ssems[b2], add=True)

        @pl.when(k >= 1)
        def _():
            pltpu.make_async_copy(rows[0], acc_s.at[didxs[0]],
                                  ssems[1 - b2]).wait()

        @pl.when(k < NCH - 2)
        def _():
            pltpu.async_copy(x_h.at[sidall_v.at[pl.ds((k + 2) * CH, CH)]],
                             rows[(b3 + 2) % 3], gsems[b2])
            pltpu.async_copy(dst_h.at[pl.ds(base + (k + 2) * CH, CH)],
                             didxs[(b3 + 2) % 3], isems[b2])

    def _six(g, carry):
        for j in range(6):
            _step(6 * g + j, j % 3, j % 2)
        return carry
    lax.fori_loop(0, NCH // 6, _six, 0)
    for k in range(NCH - NCH % 6, NCH):
        _step(k, k % 3, k % 2)
    pltpu.make_async_copy(rows[0], acc_s.at[didxs[0]],
                          ssems[(NCH - 1) % 2]).wait()

    plsc.subcore_barrier()
    _stage_out(acc_s, zbs, parts_h, cid, sid, (gsem0, gsem1),
               (ssem0, ssem1))


_SC_SCRATCH = [
    pltpu.VMEM_SHARED((N, D), jnp.float32),
    pltpu.VMEM((EP,), jnp.int32),
    pltpu.VMEM((CH,), jnp.int32),
    pltpu.VMEM((CH,), jnp.int32),
    pltpu.VMEM((CH,), jnp.int32),
    pltpu.VMEM((CH, D), jnp.float32),
    pltpu.VMEM((CH, D), jnp.float32),
    pltpu.VMEM((CH, D), jnp.float32),
    pltpu.SemaphoreType.DMA,
    pltpu.SemaphoreType.DMA,
    pltpu.SemaphoreType.DMA,
    pltpu.SemaphoreType.DMA,
    pltpu.SemaphoreType.DMA,
    pltpu.SemaphoreType.DMA,
]


def _sc_feat_deg(x, src, dst, zeros, ones):
    mesh = plsc.VectorSubcoreMesh(core_axis_name="c", subcore_axis_name="s")
    f = pl.kernel(
        functools.partial(_sc_body_feat, True),
        out_type=(jax.ShapeDtypeStruct((NC * N, D), jnp.float32),
                  jax.ShapeDtypeStruct((NC * N, D), jnp.float32)),
        mesh=mesh,
        scratch_types=_SC_SCRATCH,
    )
    return f(x, src, dst, zeros, ones)


def _sc_feat(x, src, dst, zeros):
    mesh = plsc.VectorSubcoreMesh(core_axis_name="c", subcore_axis_name="s")
    f = pl.kernel(
        functools.partial(_sc_body_feat, False),
        out_type=jax.ShapeDtypeStruct((NC * N, D), jnp.float32),
        mesh=mesh,
        scratch_types=_SC_SCRATCH,
    )
    return f(x, src, dst, zeros)


def _sage_out(parts_ref, degw_ref, x_ref, wl_ref, bl_ref, wr_ref):
    s = parts_ref[0] + parts_ref[1]
    deg = (degw_ref[0] + degw_ref[1])[:, 0:1]
    agg = s / jnp.maximum(deg, 1.0)
    return (
        lax.dot_general(agg, wl_ref[...], (((1,), (1,)), ((), ())),
                        preferred_element_type=jnp.float32)
        + lax.dot_general(x_ref[...], wr_ref[...], (((1,), (1,)), ((), ())),
                          preferred_element_type=jnp.float32)
        + bl_ref[...]
    )


def _combine_body(parts_ref, degw_ref, x_ref, wl_ref, bl_ref, wr_ref, out_ref):
    out_ref[...] = _sage_out(parts_ref, degw_ref, x_ref, wl_ref, bl_ref, wr_ref)


def _tc_combine(parts, degw, x, wl, bl, wr):
    grid = (NB,)
    return pl.pallas_call(
        _combine_body,
        grid=grid,
        in_specs=[
            pl.BlockSpec((NC, BLK, D), lambda i: (0, i, 0)),
            pl.BlockSpec((NC, BLK, D), lambda i: (0, i, 0)),
            pl.BlockSpec((BLK, D), lambda i: (i, 0)),
            pl.BlockSpec((D, D), lambda i: (0, 0)),
            pl.BlockSpec((1, D), lambda i: (0, 0)),
            pl.BlockSpec((D, D), lambda i: (0, 0)),
        ],
        out_specs=pl.BlockSpec((BLK, D), lambda i: (i, 0)),
        out_shape=jax.ShapeDtypeStruct((N, D), jnp.float32),
    )(parts, degw, x, wl, bl, wr)


def _combine_pool_body(parts_ref, degw_ref, x1_ref, wl_ref, bl_ref, wr_ref,
                       b_ref, wf1_ref, bf1_ref, wf2_ref, bf2_ref,
                       out_ref, acc_ref):
    i = pl.program_id(0)

    @pl.when(i == 0)
    def _():
        acc_ref[...] = jnp.full((G, 2 * D), -jnp.inf, jnp.float32)

    h2 = _sage_out(parts_ref, degw_ref, x1_ref, wl_ref, bl_ref, wr_ref)
    h1 = x1_ref[...]
    bv = b_ref[0]
    b_lo = jnp.min(bv)
    b_hi = jnp.max(bv)
    neg = jnp.float32(-jnp.inf)

    def _seg(b, carry):
        m = bv == b
        m1 = jnp.max(jnp.where(m, h1, neg), axis=0)
        m2 = jnp.max(jnp.where(m, h2, neg), axis=0)
        row = jnp.concatenate([m1, m2])[None, :]
        acc_ref[pl.ds(b, 1), :] = jnp.maximum(acc_ref[pl.ds(b, 1), :], row)
        return carry
    lax.fori_loop(b_lo, b_hi + 1, _seg, 0)

    @pl.when(i == NB - 1)
    def _():
        pooled = acc_ref[...]
        z = lax.dot_general(pooled, wf1_ref[...], (((1,), (1,)), ((), ())),
                            preferred_element_type=jnp.float32) + bf1_ref[...]
        z = jnp.maximum(z, 0.0)
        logits = lax.dot_general(z, wf2_ref[...], (((1,), (1,)), ((), ())),
                                 preferred_element_type=jnp.float32) + bf2_ref[...]
        mx = jnp.max(logits, axis=-1, keepdims=True)
        sh = logits - mx
        lse = jnp.log(jnp.sum(jnp.exp(sh), axis=-1, keepdims=True))
        out_ref[...] = sh - lse


def _combine_pool(parts, degw, x1, wl, bl, wr, batch3d, wf1, bf1, wf2, bf2):
    grid = (NB,)
    return pl.pallas_call(
        _combine_pool_body,
        grid=grid,
        in_specs=[
            pl.BlockSpec((NC, BLK, D), lambda i: (0, i, 0)),
            pl.BlockSpec((NC, BLK, D), lambda i: (0, i, 0)),
            pl.BlockSpec((BLK, D), lambda i: (i, 0)),
            pl.BlockSpec((D, D), lambda i: (0, 0)),
            pl.BlockSpec((1, D), lambda i: (0, 0)),
            pl.BlockSpec((D, D), lambda i: (0, 0)),
            pl.BlockSpec((1, BLK, 1), lambda i: (i, 0, 0)),
            pl.BlockSpec((D, 2 * D), lambda i: (0, 0)),
            pl.BlockSpec((1, D), lambda i: (0, 0)),
            pl.BlockSpec((C, D), lambda i: (0, 0)),
            pl.BlockSpec((1, C), lambda i: (0, 0)),
        ],
        out_specs=pl.BlockSpec((G, C), lambda i: (0, 0)),
        out_shape=jax.ShapeDtypeStruct((G, C), jnp.float32),
        scratch_shapes=[pltpu.VMEM((G, 2 * D), jnp.float32)],
    )(parts, degw, x1, wl, bl, wr, batch3d, wf1, bf1, wf2, bf2)


def kernel(x, edge_index, batch, Wl1, bl1, Wr1, Wl2, bl2, Wr2,
           Wfc1, bfc1, Wfc2, bfc2):
    src = edge_index[0]
    dst = edge_index[1]
    zeros = jnp.zeros((ZCH, D), jnp.float32)
    ones = jnp.ones((CH, D), jnp.float32)

    parts1, degw = _sc_feat_deg(x, src, dst, zeros, ones)
    degw = degw.reshape(NC, N, D)
    x1 = _tc_combine(parts1.reshape(NC, N, D), degw, x,
                     Wl1, bl1.reshape(1, D), Wr1)

    parts2 = _sc_feat(x1, src, dst, zeros)
    return _combine_pool(parts2.reshape(NC, N, D), degw, x1,
                         Wl2, bl2.reshape(1, D), Wr2,
                         batch.reshape(NB, BLK, 1),
                         Wfc1, bfc1.reshape(1, D), Wfc2, bfc2.reshape(1, C))


# speedup vs baseline: 11.8867x; 1.0240x over previous
import functools

import jax
import jax.numpy as jnp
from jax import lax
from jax.experimental import pallas as pl
from jax.experimental.pallas import tpu as pltpu
from jax.experimental.pallas import tpu_sc as plsc

N = 10000
E = 320000
D = 128
C = 10
G = 64

NC = 2
NS = 16
NW = NC * NS
EP = E // NW
CH = 80
NCH = EP // CH

WSTR = 624
TAIL = 16
TOFF = NS * WSTR
ZCH = 48
NZC = WSTR // ZCH

BLK = 2000
NB = N // BLK
DW = 16


def _stage_zero(z_h, zb_v, acc_s, sid, sem):
    pltpu.sync_copy(z_h, zb_v)
    for k in range(NZC):
        pltpu.async_copy(zb_v, acc_s.at[pl.ds(sid * WSTR + k * ZCH, ZCH)],
                         sem)
    for k in range(NZC):
        pltpu.make_async_copy(zb_v, acc_s.at[pl.ds(sid * WSTR, ZCH)],
                              sem).wait()

    @pl.when(sid == 0)
    def _():
        pltpu.sync_copy(zb_v.at[pl.ds(0, TAIL)], acc_s.at[pl.ds(TOFF, TAIL)])


def _stage_out(acc_s, zbs, out_h, cid, sid, rsems, wsems):
    for k in range(NZC):
        p = k % 2
        r0 = sid * WSTR + k * ZCH
        if k >= 2:
            pltpu.make_async_copy(zbs[p], out_h.at[pl.ds(cid * N, ZCH)],
                                  wsems[p]).wait()
        pltpu.async_copy(acc_s.at[pl.ds(r0, ZCH)], zbs[p], rsems[p])
        pltpu.make_async_copy(acc_s.at[pl.ds(r0, ZCH)], zbs[p],
                              rsems[p]).wait()
        pltpu.async_copy(zbs[p], out_h.at[pl.ds(cid * N + r0, ZCH)],
                         wsems[p])
    for p in ((NZC - 2) % 2, (NZC - 1) % 2):
        pltpu.make_async_copy(zbs[p], out_h.at[pl.ds(cid * N, ZCH)],
                              wsems[p]).wait()

    @pl.when(sid == 0)
    def _():
        pltpu.sync_copy(acc_s.at[pl.ds(TOFF, TAIL)],
                        zbs[0].at[pl.ds(0, TAIL)])
        pltpu.sync_copy(zbs[0].at[pl.ds(0, TAIL)],
                        out_h.at[pl.ds(cid * N + TOFF, TAIL)])


def _sc_body_feat(with_deg, *refs):
    if with_deg:
        (x_h, src_h, dst_h, zeros_h, ones_h, parts_h, degw_h,
         acc_s, sidall_v, didx0_v, didx1_v, didx2_v,
         rows0_v, rows1_v, rows2_v,
         gsem0, gsem1, ssem0, ssem1, isem0, isem1) = refs
    else:
        (x_h, src_h, dst_h, zeros_h, parts_h,
         acc_s, sidall_v, didx0_v, didx1_v, didx2_v,
         rows0_v, rows1_v, rows2_v,
         gsem0, gsem1, ssem0, ssem1, isem0, isem1) = refs
    cid = lax.axis_index("c")
    sid = lax.axis_index("s")
    wid = cid * NS + sid

    zb_v = rows1_v.at[pl.ds(0, ZCH)]
    zbs = (zb_v, rows2_v.at[pl.ds(0, ZCH)])
    _stage_zero(zeros_h, zb_v, acc_s, sid, isem0)
    pltpu.sync_copy(src_h.at[pl.ds(wid * EP, EP)], sidall_v)
    if with_deg:
        pltpu.sync_copy(ones_h, rows0_v)
    plsc.subcore_barrier()

    didxs = (didx0_v, didx1_v, didx2_v)
    ssems = (ssem0, ssem1)
    isems = (isem0, isem1)
    dbase = wid * EP

    if with_deg:
        for b in range(2):
            pltpu.async_copy(dst_h.at[pl.ds(dbase + b * CH, CH)], didxs[b],
                             isems[b])

        def _dstep(k, b3, b2):
            pltpu.make_async_copy(dst_h.at[pl.ds(0, CH)], didxs[b3],
                                  isems[b2]).wait()
            pltpu.async_copy(rows0_v, acc_s.at[didxs[b3]], ssems[b2],
                             add=True)

            @pl.when(k >= 1)
            def _():
                pltpu.make_async_copy(rows0_v, acc_s.at[didxs[0]],
                                      ssems[1 - b2]).wait()

            @pl.when(k < NCH - 2)
            def _():
                pltpu.async_copy(dst_h.at[pl.ds(dbase + (k + 2) * CH, CH)],
                                 didxs[(b3 + 2) % 3], isems[b2])

        def _dsix(g, carry):
            for j in range(6):
                _dstep(6 * g + j, j % 3, j % 2)
            return carry
        lax.fori_loop(0, NCH // 6, _dsix, 0)
        for k in range(NCH - NCH % 6, NCH):
            _dstep(k, k % 3, k % 2)
        pltpu.make_async_copy(rows0_v, acc_s.at[didxs[0]],
                              ssems[(NCH - 1) % 2]).wait()

        plsc.subcore_barrier()
        _stage_out(acc_s, zbs, degw_h, cid, sid, (gsem0, gsem1),
                   (ssem0, ssem1))
        _stage_zero(zeros_h, zb_v, acc_s, sid, isem0)
        plsc.subcore_barrier()

    rows = (rows0_v, rows1_v, rows2_v)
    gsems = (gsem0, gsem1)
    base = wid * EP

    for b in range(2):
        pltpu.async_copy(dst_h.at[pl.ds(base + b * CH, CH)], didxs[b],
                         isems[b])
        pltpu.async_copy(x_h.at[sidall_v.at[pl.ds(b * CH, CH)]],
                         rows[b], gsems[b])

    def _step(k, b3, b2):
        pltpu.make_async_copy(dst_h.at[pl.ds(0, CH)], didxs[b3],
                              isems[b2]).wait()
        pltpu.make_async_copy(x_h.at[sidall_v.at[pl.ds(0, CH)]],
                              rows[b3], gsems[b2]).wait()
        pltpu.async_copy(rows[b3], acc_s.at[didxs[b3]], ssems[b2], add=True)

        @pl.when(k >= 1)
        def _():
            pltpu.make_async_copy(rows[0], acc_s.at[didxs[0]],
                                  ssems[1 - b2]).wait()

        @pl.when(k < NCH - 2)
        def _():
            pltpu.async_copy(x_h.at[sidall_v.at[pl.ds((k + 2) * CH, CH)]],
                             rows[(b3 + 2) % 3], gsems[b2])
            pltpu.async_copy(dst_h.at[pl.ds(base + (k + 2) * CH, CH)],
                             didxs[(b3 + 2) % 3], isems[b2])

    def _six(g, carry):
        for j in range(6):
            _step(6 * g + j, j % 3, j % 2)
        return carry
    lax.fori_loop(0, NCH // 6, _six, 0)
    for k in range(NCH - NCH % 6, NCH):
        _step(k, k % 3, k % 2)
    pltpu.make_async_copy(rows[0], acc_s.at[didxs[0]],
                          ssems[(NCH - 1) % 2]).wait()

    plsc.subcore_barrier()
    _stage_out(acc_s, zbs, parts_h, cid, sid, (gsem0, gsem1),
               (ssem0, ssem1))


_SC_SCRATCH = [
    pltpu.VMEM_SHARED((N, D), jnp.float32),
    pltpu.VMEM((EP,), jnp.int32),
    pltpu.VMEM((CH,), jnp.int32),
    pltpu.VMEM((CH,), jnp.int32),
    pltpu.VMEM((CH,), jnp.int32),
    pltpu.VMEM((CH, D), jnp.float32),
    pltpu.VMEM((CH, D), jnp.float32),
    pltpu.VMEM((CH, D), jnp.float32),
    pltpu.SemaphoreType.DMA,
    pltpu.SemaphoreType.DMA,
    pltpu.SemaphoreType.DMA,
    pltpu.SemaphoreType.DMA,
    pltpu.SemaphoreType.DMA,
    pltpu.SemaphoreType.DMA,
]


def _sc_feat_deg(x, src, dst, zeros, ones):
    mesh = plsc.VectorSubcoreMesh(core_axis_name="c", subcore_axis_name="s")
    f = pl.kernel(
        functools.partial(_sc_body_feat, True),
        out_type=(jax.ShapeDtypeStruct((NC * N, D), jnp.float32),
                  jax.ShapeDtypeStruct((NC * N, D), jnp.float32)),
        mesh=mesh,
        scratch_types=_SC_SCRATCH,
    )
    return f(x, src, dst, zeros, ones)


def _sc_feat(x, src, dst, zeros):
    mesh = plsc.VectorSubcoreMesh(core_axis_name="c", subcore_axis_name="s")
    f = pl.kernel(
        functools.partial(_sc_body_feat, False),
        out_type=jax.ShapeDtypeStruct((NC * N, D), jnp.float32),
        mesh=mesh,
        scratch_types=_SC_SCRATCH,
    )
    return f(x, src, dst, zeros)


def _sage_out(parts_ref, degw_ref, x_ref, wl_ref, bl_ref, wr_ref):
    s = parts_ref[0] + parts_ref[1]
    deg = (degw_ref[0] + degw_ref[1])[:, 0:1]
    agg = s / jnp.maximum(deg, 1.0)
    return (
        lax.dot_general(agg, wl_ref[...], (((1,), (1,)), ((), ())),
                        preferred_element_type=jnp.float32)
        + lax.dot_general(x_ref[...], wr_ref[...], (((1,), (1,)), ((), ())),
                          preferred_element_type=jnp.float32)
        + bl_ref[...]
    )


def _combine_body(parts_ref, degw_ref, x_ref, wl_ref, bl_ref, wr_ref, out_ref):
    out_ref[...] = _sage_out(parts_ref, degw_ref, x_ref, wl_ref, bl_ref, wr_ref)


def _tc_combine(parts, degw, x, wl, bl, wr):
    grid = (NB,)
    return pl.pallas_call(
        _combine_body,
        grid=grid,
        in_specs=[
            pl.BlockSpec((NC, BLK, D), lambda i: (0, i, 0)),
            pl.BlockSpec((NC, BLK, D), lambda i: (0, i, 0)),
            pl.BlockSpec((BLK, D), lambda i: (i, 0)),
            pl.BlockSpec((D, D), lambda i: (0, 0)),
            pl.BlockSpec((1, D), lambda i: (0, 0)),
            pl.BlockSpec((D, D), lambda i: (0, 0)),
        ],
        out_specs=pl.BlockSpec((BLK, D), lambda i: (i, 0)),
        out_shape=jax.ShapeDtypeStruct((N, D), jnp.float32),
    )(parts, degw, x, wl, bl, wr)


def _combine_pool_body(parts_ref, degw_ref, x1_ref, wl_ref, bl_ref, wr_ref,
                       b_ref, wf1_ref, bf1_ref, wf2_ref, bf2_ref,
                       out_ref, acc_ref):
    i = pl.program_id(0)

    @pl.when(i == 0)
    def _():
        acc_ref[...] = jnp.full((G, 2 * D), -jnp.inf, jnp.float32)

    h2 = _sage_out(parts_ref, degw_ref, x1_ref, wl_ref, bl_ref, wr_ref)
    h1 = x1_ref[...]
    bv = b_ref[0]
    neg = jnp.float32(-jnp.inf)

    SB = 400
    for sb in range(BLK // SB):
        bvs = bv[sb * SB:(sb + 1) * SB]
        h1s = h1[sb * SB:(sb + 1) * SB]
        h2s = h2[sb * SB:(sb + 1) * SB]
        b_lo = jnp.min(bvs)
        b_hi = jnp.max(bvs)

        def _seg(b, carry, bvs=bvs, h1s=h1s, h2s=h2s):
            m = bvs == b
            m1 = jnp.max(jnp.where(m, h1s, neg), axis=0)
            m2 = jnp.max(jnp.where(m, h2s, neg), axis=0)
            row = jnp.concatenate([m1, m2])[None, :]
            acc_ref[pl.ds(b, 1), :] = jnp.maximum(acc_ref[pl.ds(b, 1), :],
                                                  row)
            return carry
        lax.fori_loop(b_lo, b_hi + 1, _seg, 0)

    @pl.when(i == NB - 1)
    def _():
        pooled = acc_ref[...]
        z = lax.dot_general(pooled, wf1_ref[...], (((1,), (1,)), ((), ())),
                            preferred_element_type=jnp.float32) + bf1_ref[...]
        z = jnp.maximum(z, 0.0)
        logits = lax.dot_general(z, wf2_ref[...], (((1,), (1,)), ((), ())),
                                 preferred_element_type=jnp.float32) + bf2_ref[...]
        mx = jnp.max(logits, axis=-1, keepdims=True)
        sh = logits - mx
        lse = jnp.log(jnp.sum(jnp.exp(sh), axis=-1, keepdims=True))
        out_ref[...] = sh - lse


def _combine_pool(parts, degw, x1, wl, bl, wr, batch3d, wf1, bf1, wf2, bf2):
    grid = (NB,)
    return pl.pallas_call(
        _combine_pool_body,
        grid=grid,
        in_specs=[
            pl.BlockSpec((NC, BLK, D), lambda i: (0, i, 0)),
            pl.BlockSpec((NC, BLK, D), lambda i: (0, i, 0)),
            pl.BlockSpec((BLK, D), lambda i: (i, 0)),
            pl.BlockSpec((D, D), lambda i: (0, 0)),
            pl.BlockSpec((1, D), lambda i: (0, 0)),
            pl.BlockSpec((D, D), lambda i: (0, 0)),
            pl.BlockSpec((1, BLK, 1), lambda i: (i, 0, 0)),
            pl.BlockSpec((D, 2 * D), lambda i: (0, 0)),
            pl.BlockSpec((1, D), lambda i: (0, 0)),
            pl.BlockSpec((C, D), lambda i: (0, 0)),
            pl.BlockSpec((1, C), lambda i: (0, 0)),
        ],
        out_specs=pl.BlockSpec((G, C), lambda i: (0, 0)),
        out_shape=jax.ShapeDtypeStruct((G, C), jnp.float32),
        scratch_shapes=[pltpu.VMEM((G, 2 * D), jnp.float32)],
    )(parts, degw, x1, wl, bl, wr, batch3d, wf1, bf1, wf2, bf2)


def kernel(x, edge_index, batch, Wl1, bl1, Wr1, Wl2, bl2, Wr2,
           Wfc1, bfc1, Wfc2, bfc2):
    src = edge_index[0]
    dst = edge_index[1]
    zeros = jnp.zeros((ZCH, D), jnp.float32)
    ones = jnp.ones((CH, D), jnp.float32)

    parts1, degw = _sc_feat_deg(x, src, dst, zeros, ones)
    degw = degw.reshape(NC, N, D)
    x1 = _tc_combine(parts1.reshape(NC, N, D), degw, x,
                     Wl1, bl1.reshape(1, D), Wr1)

    parts2 = _sc_feat(x1, src, dst, zeros)
    return _combine_pool(parts2.reshape(NC, N, D), degw, x1,
                         Wl2, bl2.reshape(1, D), Wr2,
                         batch.reshape(NB, BLK, 1),
                         Wfc1, bfc1.reshape(1, D), Wfc2, bfc2.reshape(1, C))
